# baseline (speedup 1.0000x reference)
import dataclasses
import functools

import jax
import jax.numpy as jnp
from jax import lax
from jax.experimental import pallas as pl
from jax.experimental.pallas import tpu as pltpu
from jax.experimental.pallas import tpu_sc as plsc

BATCH = 16384
EMBED_DIM = 64
NUM_CORES = 2
NUM_SUBCORES = 16
NUM_WORKERS = NUM_CORES * NUM_SUBCORES
ROWS_PER_WORKER = BATCH // NUM_WORKERS
WAVE = 128
NUM_WAVES = ROWS_PER_WORKER // WAVE
LANES = 16
DBLK = EMBED_DIM // LANES
GROUPS = WAVE // LANES
PAIR_ROWS = 1000000 // 2


def _gmf_kernel(hu_hbm, hi_hbm, pu_hbm, pi_hbm, tu_hbm, ti_hbm, w_hbm, b_hbm,
                out_hbm, idx_u, idx_i, par_u, par_i, rows_u, rows_i, wv, bv,
                out_v, sem0, sem1):
    wid = lax.axis_index("s") * NUM_CORES + lax.axis_index("c")
    base = wid * ROWS_PER_WORKER

    pltpu.sync_copy(hu_hbm.at[wid], idx_u)
    pltpu.sync_copy(hi_hbm.at[wid], idx_i)
    pltpu.sync_copy(pu_hbm.at[wid], par_u)
    pltpu.sync_copy(pi_hbm.at[wid], par_i)
    pltpu.sync_copy(w_hbm, wv)
    pltpu.sync_copy(b_hbm, bv)

    w_regs = [wv[pl.ds(d * LANES, LANES)] for d in range(DBLK)]
    b_vec = bv[...]
    lane = lax.iota(jnp.int32, LANES)
    sems = (sem0, sem1)

    def fire(w):
        slot = w % 2
        return [
            pltpu.async_copy(tu_hbm.at[idx_u.at[w]], rows_u.at[slot], sems[slot]),
            pltpu.async_copy(ti_hbm.at[idx_i.at[w]], rows_i.at[slot], sems[slot]),
        ]

    def compute(w):
        slot = w % 2

        @pl.loop(0, GROUPS)
        def _(g):
            pu16 = par_u[w, pl.ds(g * LANES, LANES)]
            pi16 = par_i[w, pl.ds(g * LANES, LANES)]
            res = b_vec
            for k in range(LANES):
                r = g * LANES + k
                off_u = pu16[k] * EMBED_DIM
                off_i = pi16[k] * EMBED_DIM
                acc = (rows_u[slot, r, pl.ds(off_u, LANES)]
                       * rows_i[slot, r, pl.ds(off_i, LANES)] * w_regs[0])
                for d in range(1, DBLK):
                    acc = acc + (rows_u[slot, r, pl.ds(off_u + d * LANES, LANES)]
                                 * rows_i[slot, r, pl.ds(off_i + d * LANES, LANES)]
                                 * w_regs[d])
                res = jnp.where(lane == k, res + jnp.sum(acc), res)
            out_v[pl.ds(w * WAVE + g * LANES, LANES)] = res

    pending = fire(0)
    for w in range(NUM_WAVES):
        nxt = fire(w + 1) if w + 1 < NUM_WAVES else []
        for c in pending:
            c.wait()
        compute(w)
        pending = nxt

    pltpu.sync_copy(out_v, out_hbm.at[pl.ds(base, ROWS_PER_WORKER)])


@jax.jit
def kernel(user, item, embed_user_w, embed_item_w, W, b):
    user = user.astype(jnp.int32)
    item = item.astype(jnp.int32)
    shp = (NUM_WORKERS, NUM_WAVES, WAVE)
    half_u = (user >> 1).reshape(shp)
    half_i = (item >> 1).reshape(shp)
    par_u = (user & 1).reshape(shp)
    par_i = (item & 1).reshape(shp)
    tab_u = embed_user_w.reshape(PAIR_ROWS, 2 * EMBED_DIM)
    tab_i = embed_item_w.reshape(PAIR_ROWS, 2 * EMBED_DIM)
    w_flat = W.reshape(EMBED_DIM)
    b_pad = jnp.broadcast_to(b, (LANES,))

    mesh = plsc.VectorSubcoreMesh(core_axis_name="c", subcore_axis_name="s")
    cp = pltpu.CompilerParams()
    if "needs_layout_passes" in pltpu.CompilerParams.__dataclass_fields__:
        cp = dataclasses.replace(cp, needs_layout_passes=False)
    run = pl.kernel(
        _gmf_kernel,
        out_type=jax.ShapeDtypeStruct((BATCH,), jnp.float32),
        mesh=mesh,
        compiler_params=cp,
        scratch_types=[
            pltpu.VMEM((NUM_WAVES, WAVE), jnp.int32),
            pltpu.VMEM((NUM_WAVES, WAVE), jnp.int32),
            pltpu.VMEM((NUM_WAVES, WAVE), jnp.int32),
            pltpu.VMEM((NUM_WAVES, WAVE), jnp.int32),
            pltpu.VMEM((2, WAVE, 2 * EMBED_DIM), jnp.float32),
            pltpu.VMEM((2, WAVE, 2 * EMBED_DIM), jnp.float32),
            pltpu.VMEM((EMBED_DIM,), jnp.float32),
            pltpu.VMEM((LANES,), jnp.float32),
            pltpu.VMEM((ROWS_PER_WORKER,), jnp.float32),
            pltpu.SemaphoreType.DMA,
            pltpu.SemaphoreType.DMA,
        ],
    )
    return run(half_u, half_i, par_u, par_i, tab_u, tab_i, w_flat, b_pad)


# speedup vs baseline: 2.7161x; 2.7161x over previous
import dataclasses

import jax
import jax.numpy as jnp
import numpy as np
from jax import lax
from jax.experimental import pallas as pl
from jax.experimental.pallas import tpu as pltpu
from jax.experimental.pallas import tpu_sc as plsc

BATCH = 16384
EMBED_DIM = 64
VOCAB = 1000000
NUM_CORES = 2
NUM_SUBCORES = 16
NUM_WORKERS = NUM_CORES * NUM_SUBCORES
ROWS_PER_WORKER = BATCH // NUM_WORKERS
WAVE = 128
NUM_WAVES = ROWS_PER_WORKER // WAVE
LANES = 16
DBLK = EMBED_DIM // LANES
GROUPS = WAVE // LANES
PACK_BLK = 4096
PACK_HALF = PACK_BLK // 2
PACK_STEPS = -(-VOCAB // PACK_BLK)
PACK_ROWS = PACK_STEPS * PACK_HALF
HIGH_MASK = np.uint32(0xFFFF0000)


def _pack_kernel(u_ref, i_ref, o_ref):
    ub = lax.bitcast_convert_type(
        u_ref[...].astype(jnp.bfloat16), jnp.uint16).astype(jnp.uint32)
    ib = lax.bitcast_convert_type(
        i_ref[...].astype(jnp.bfloat16), jnp.uint16).astype(jnp.uint32)
    packed = lax.bitcast_convert_type((ub << 16) | ib, jnp.float32)
    t = packed.T
    o_ref[...] = jnp.concatenate(
        [t[:PACK_HALF], t[PACK_HALF:]], axis=1)


def _pack_tables(ut, it):
    return pl.pallas_call(
        _pack_kernel,
        grid=(PACK_STEPS,),
        in_specs=[
            pl.BlockSpec((EMBED_DIM, PACK_BLK), lambda j: (0, j)),
            pl.BlockSpec((EMBED_DIM, PACK_BLK), lambda j: (0, j)),
        ],
        out_specs=pl.BlockSpec((PACK_HALF, 2 * EMBED_DIM),
                               lambda j: (j, 0)),
        out_shape=jax.ShapeDtypeStruct((PACK_ROWS, 2 * EMBED_DIM),
                                       jnp.float32),
        compiler_params=pltpu.CompilerParams(
            dimension_semantics=("arbitrary",)),
    )(ut, it)


def _gmf_kernel(hu_hbm, hi_hbm, pu_hbm, pi_hbm, tab_hbm, w_hbm, b_hbm,
                out_hbm, idx_u, idx_i, par_u, par_i, rows_u, rows_i, wv, bv,
                out_v, sem0, sem1):
    wid = lax.axis_index("s") * NUM_CORES + lax.axis_index("c")
    base = wid * ROWS_PER_WORKER

    pltpu.sync_copy(hu_hbm.at[wid], idx_u)
    pltpu.sync_copy(hi_hbm.at[wid], idx_i)
    pltpu.sync_copy(pu_hbm.at[wid], par_u)
    pltpu.sync_copy(pi_hbm.at[wid], par_i)
    pltpu.sync_copy(w_hbm, wv)
    pltpu.sync_copy(b_hbm, bv)

    w_regs = [wv[pl.ds(d * LANES, LANES)] for d in range(DBLK)]
    b_vec = bv[...]
    lane = lax.iota(jnp.int32, LANES)
    sems = (sem0, sem1)

    def fire(w):
        slot = w % 2
        return [
            pltpu.async_copy(tab_hbm.at[idx_u.at[w]], rows_u.at[slot], sems[slot]),
            pltpu.async_copy(tab_hbm.at[idx_i.at[w]], rows_i.at[slot], sems[slot]),
        ]

    def user_half(x):
        bits = plsc.bitcast(x, jnp.uint32)
        return plsc.bitcast(bits & HIGH_MASK, jnp.float32)

    def item_half(x):
        bits = plsc.bitcast(x, jnp.uint32)
        return plsc.bitcast(bits << 16, jnp.float32)

    def compute(w):
        slot = w % 2

        @pl.loop(0, GROUPS)
        def _(g):
            pu16 = par_u[w, pl.ds(g * LANES, LANES)]
            pi16 = par_i[w, pl.ds(g * LANES, LANES)]
            res = b_vec
            for k in range(LANES):
                r = g * LANES + k
                off_u = pu16[k] * EMBED_DIM
                off_i = pi16[k] * EMBED_DIM
                acc = (user_half(rows_u[slot, r, pl.ds(off_u, LANES)])
                       * item_half(rows_i[slot, r, pl.ds(off_i, LANES)])
                       * w_regs[0])
                for d in range(1, DBLK):
                    acc = acc + (
                        user_half(rows_u[slot, r, pl.ds(off_u + d * LANES, LANES)])
                        * item_half(rows_i[slot, r, pl.ds(off_i + d * LANES, LANES)])
                        * w_regs[d])
                res = jnp.where(lane == k, res + jnp.sum(acc), res)
            out_v[pl.ds(w * WAVE + g * LANES, LANES)] = res

    pending = fire(0)
    for w in range(NUM_WAVES):
        nxt = fire(w + 1) if w + 1 < NUM_WAVES else []
        for c in pending:
            c.wait()
        compute(w)
        pending = nxt

    pltpu.sync_copy(out_v, out_hbm.at[pl.ds(base, ROWS_PER_WORKER)])


@jax.jit
def kernel(user, item, embed_user_w, embed_item_w, W, b):
    user = user.astype(jnp.int32)
    item = item.astype(jnp.int32)
    shp = (NUM_WORKERS, NUM_WAVES, WAVE)

    def to_packed(r):
        q = r % PACK_BLK
        row = (r // PACK_BLK) * PACK_HALF + (q % PACK_HALF)
        half = q // PACK_HALF
        return row.reshape(shp), half.reshape(shp)

    half_u, par_u = to_packed(user)
    half_i, par_i = to_packed(item)
    tab = _pack_tables(embed_user_w.T, embed_item_w.T)
    w_flat = W.reshape(EMBED_DIM)
    b_pad = jnp.broadcast_to(b, (LANES,))

    mesh = plsc.VectorSubcoreMesh(core_axis_name="c", subcore_axis_name="s")
    cp = pltpu.CompilerParams()
    if "needs_layout_passes" in pltpu.CompilerParams.__dataclass_fields__:
        cp = dataclasses.replace(cp, needs_layout_passes=False)
    run = pl.kernel(
        _gmf_kernel,
        out_type=jax.ShapeDtypeStruct((BATCH,), jnp.float32),
        mesh=mesh,
        compiler_params=cp,
        scratch_types=[
            pltpu.VMEM((NUM_WAVES, WAVE), jnp.int32),
            pltpu.VMEM((NUM_WAVES, WAVE), jnp.int32),
            pltpu.VMEM((NUM_WAVES, WAVE), jnp.int32),
            pltpu.VMEM((NUM_WAVES, WAVE), jnp.int32),
            pltpu.VMEM((2, WAVE, 2 * EMBED_DIM), jnp.float32),
            pltpu.VMEM((2, WAVE, 2 * EMBED_DIM), jnp.float32),
            pltpu.VMEM((EMBED_DIM,), jnp.float32),
            pltpu.VMEM((LANES,), jnp.float32),
            pltpu.VMEM((ROWS_PER_WORKER,), jnp.float32),
            pltpu.SemaphoreType.DMA,
            pltpu.SemaphoreType.DMA,
        ],
    )
    return run(half_u, half_i, par_u, par_i, tab, w_flat, b_pad)


# speedup vs baseline: 3.3232x; 1.2235x over previous
import dataclasses

import jax
import jax.numpy as jnp
import numpy as np
from jax import lax
from jax.experimental import pallas as pl
from jax.experimental.pallas import tpu as pltpu
from jax.experimental.pallas import tpu_sc as plsc

BATCH = 16384
EMBED_DIM = 64
VOCAB = 1000000
NUM_CORES = 2
NUM_SUBCORES = 16
NUM_WORKERS = NUM_CORES * NUM_SUBCORES
ROWS_PER_WORKER = BATCH // NUM_WORKERS
WAVE = 128
NUM_WAVES = ROWS_PER_WORKER // WAVE
LANES = 16
DBLK = EMBED_DIM // LANES
GROUPS = WAVE // LANES
PACK_BLK = 8192
PACK_HALF = PACK_BLK // 2
PACK_STEPS = -(-VOCAB // PACK_BLK)
PACK_ROWS = PACK_STEPS * PACK_HALF
HIGH_MASK = np.uint32(0xFFFF0000)


def _pack_kernel(u_ref, i_ref, o_ref):
    ub = lax.bitcast_convert_type(
        u_ref[...].astype(jnp.bfloat16), jnp.uint16).astype(jnp.uint32)
    ib = lax.bitcast_convert_type(
        i_ref[...].astype(jnp.bfloat16), jnp.uint16).astype(jnp.uint32)
    packed = lax.bitcast_convert_type((ub << 16) | ib, jnp.float32)
    t = packed.T
    o_ref[...] = jnp.concatenate(
        [t[:PACK_HALF], t[PACK_HALF:]], axis=1)


def _pack_tables(ut, it):
    return pl.pallas_call(
        _pack_kernel,
        grid=(PACK_STEPS,),
        in_specs=[
            pl.BlockSpec((EMBED_DIM, PACK_BLK), lambda j: (0, j)),
            pl.BlockSpec((EMBED_DIM, PACK_BLK), lambda j: (0, j)),
        ],
        out_specs=pl.BlockSpec((PACK_HALF, 2 * EMBED_DIM),
                               lambda j: (j, 0)),
        out_shape=jax.ShapeDtypeStruct((PACK_ROWS, 2 * EMBED_DIM),
                                       jnp.float32),
        compiler_params=pltpu.CompilerParams(
            dimension_semantics=("arbitrary",)),
    )(ut, it)


def _gmf_kernel(hu_hbm, hi_hbm, pu_hbm, pi_hbm, tab_hbm, w_hbm, b_hbm,
                out_hbm, idx_u, idx_i, par_u, par_i, rows_u, rows_i, wv, bv,
                out_v, sem0, sem1):
    wid = lax.axis_index("s") * NUM_CORES + lax.axis_index("c")
    base = wid * ROWS_PER_WORKER

    pltpu.sync_copy(hu_hbm.at[wid], idx_u)
    pltpu.sync_copy(hi_hbm.at[wid], idx_i)
    pltpu.sync_copy(pu_hbm.at[wid], par_u)
    pltpu.sync_copy(pi_hbm.at[wid], par_i)
    pltpu.sync_copy(w_hbm, wv)
    pltpu.sync_copy(b_hbm, bv)

    w_regs = [wv[pl.ds(d * LANES, LANES)] for d in range(DBLK)]
    b_vec = bv[...]
    lane = lax.iota(jnp.int32, LANES)
    sems = (sem0, sem1)

    def fire(w):
        slot = w % 2
        return [
            pltpu.async_copy(tab_hbm.at[idx_u.at[w]], rows_u.at[slot], sems[slot]),
            pltpu.async_copy(tab_hbm.at[idx_i.at[w]], rows_i.at[slot], sems[slot]),
        ]

    def user_half(x):
        bits = plsc.bitcast(x, jnp.uint32)
        return plsc.bitcast(bits & HIGH_MASK, jnp.float32)

    def item_half(x):
        bits = plsc.bitcast(x, jnp.uint32)
        return plsc.bitcast(bits << 16, jnp.float32)

    def compute(w):
        slot = w % 2

        @pl.loop(0, GROUPS)
        def _(g):
            pu16 = par_u[w, pl.ds(g * LANES, LANES)]
            pi16 = par_i[w, pl.ds(g * LANES, LANES)]
            res = b_vec
            for k in range(LANES):
                r = g * LANES + k
                off_u = pu16[k] * EMBED_DIM
                off_i = pi16[k] * EMBED_DIM
                acc = (user_half(rows_u[slot, r, pl.ds(off_u, LANES)])
                       * item_half(rows_i[slot, r, pl.ds(off_i, LANES)])
                       * w_regs[0])
                for d in range(1, DBLK):
                    acc = acc + (
                        user_half(rows_u[slot, r, pl.ds(off_u + d * LANES, LANES)])
                        * item_half(rows_i[slot, r, pl.ds(off_i + d * LANES, LANES)])
                        * w_regs[d])
                res = jnp.where(lane == k, res + jnp.sum(acc), res)
            out_v[pl.ds(w * WAVE + g * LANES, LANES)] = res

    pending = fire(0)
    for w in range(NUM_WAVES):
        nxt = fire(w + 1) if w + 1 < NUM_WAVES else []
        for c in pending:
            c.wait()
        compute(w)
        pending = nxt

    pltpu.sync_copy(out_v, out_hbm.at[pl.ds(base, ROWS_PER_WORKER)])


@jax.jit
def kernel(user, item, embed_user_w, embed_item_w, W, b):
    user = user.astype(jnp.int32)
    item = item.astype(jnp.int32)
    shp = (NUM_WORKERS, NUM_WAVES, WAVE)

    def to_packed(r):
        q = r % PACK_BLK
        row = (r // PACK_BLK) * PACK_HALF + (q % PACK_HALF)
        half = q // PACK_HALF
        return row.reshape(shp), half.reshape(shp)

    half_u, par_u = to_packed(user)
    half_i, par_i = to_packed(item)
    tab = _pack_tables(embed_user_w.T, embed_item_w.T)
    w_flat = W.reshape(EMBED_DIM)
    b_pad = jnp.broadcast_to(b, (LANES,))

    mesh = plsc.VectorSubcoreMesh(core_axis_name="c", subcore_axis_name="s")
    cp = pltpu.CompilerParams()
    if "needs_layout_passes" in pltpu.CompilerParams.__dataclass_fields__:
        cp = dataclasses.replace(cp, needs_layout_passes=False)
    run = pl.kernel(
        _gmf_kernel,
        out_type=jax.ShapeDtypeStruct((BATCH,), jnp.float32),
        mesh=mesh,
        compiler_params=cp,
        scratch_types=[
            pltpu.VMEM((NUM_WAVES, WAVE), jnp.int32),
            pltpu.VMEM((NUM_WAVES, WAVE), jnp.int32),
            pltpu.VMEM((NUM_WAVES, WAVE), jnp.int32),
            pltpu.VMEM((NUM_WAVES, WAVE), jnp.int32),
            pltpu.VMEM((2, WAVE, 2 * EMBED_DIM), jnp.float32),
            pltpu.VMEM((2, WAVE, 2 * EMBED_DIM), jnp.float32),
            pltpu.VMEM((EMBED_DIM,), jnp.float32),
            pltpu.VMEM((LANES,), jnp.float32),
            pltpu.VMEM((ROWS_PER_WORKER,), jnp.float32),
            pltpu.SemaphoreType.DMA,
            pltpu.SemaphoreType.DMA,
        ],
    )
    return run(half_u, half_i, par_u, par_i, tab, w_flat, b_pad)


# speedup vs baseline: 3.7098x; 1.1163x over previous
import dataclasses

import jax
import jax.numpy as jnp
import numpy as np
from jax import lax
from jax.experimental import pallas as pl
from jax.experimental.pallas import tpu as pltpu
from jax.experimental.pallas import tpu_sc as plsc

BATCH = 16384
EMBED_DIM = 64
VOCAB = 1000000
NUM_CORES = 2
NUM_SUBCORES = 16
NUM_WORKERS = NUM_CORES * NUM_SUBCORES
ROWS_PER_WORKER = BATCH // NUM_WORKERS
WAVE = 128
NUM_WAVES = ROWS_PER_WORKER // WAVE
LANES = 16
DBLK = EMBED_DIM // LANES
GROUPS = WAVE // LANES
PACK_BLK = 16384
PACK_HALF = PACK_BLK // 2
PACK_STEPS = -(-VOCAB // PACK_BLK)
PACK_ROWS = PACK_STEPS * PACK_HALF
HIGH_MASK = np.uint32(0xFFFF0000)


def _pack_kernel(u_ref, i_ref, o_ref):
    ub = lax.bitcast_convert_type(
        u_ref[...].astype(jnp.bfloat16), jnp.uint16).astype(jnp.uint32)
    ib = lax.bitcast_convert_type(
        i_ref[...].astype(jnp.bfloat16), jnp.uint16).astype(jnp.uint32)
    packed = lax.bitcast_convert_type((ub << 16) | ib, jnp.float32)
    t = packed.T
    o_ref[...] = jnp.concatenate(
        [t[:PACK_HALF], t[PACK_HALF:]], axis=1)


def _pack_tables(ut, it):
    return pl.pallas_call(
        _pack_kernel,
        grid=(PACK_STEPS,),
        in_specs=[
            pl.BlockSpec((EMBED_DIM, PACK_BLK), lambda j: (0, j)),
            pl.BlockSpec((EMBED_DIM, PACK_BLK), lambda j: (0, j)),
        ],
        out_specs=pl.BlockSpec((PACK_HALF, 2 * EMBED_DIM),
                               lambda j: (j, 0)),
        out_shape=jax.ShapeDtypeStruct((PACK_ROWS, 2 * EMBED_DIM),
                                       jnp.float32),
        compiler_params=pltpu.CompilerParams(
            dimension_semantics=("arbitrary",)),
    )(ut, it)


def _gmf_kernel(hu_hbm, hi_hbm, pu_hbm, pi_hbm, tab_hbm, w_hbm, b_hbm,
                out_hbm, idx_u, idx_i, par_u, par_i, rows_u, rows_i, wv, bv,
                out_v, sem0, sem1):
    wid = lax.axis_index("s") * NUM_CORES + lax.axis_index("c")
    base = wid * ROWS_PER_WORKER

    pltpu.sync_copy(hu_hbm.at[wid], idx_u)
    pltpu.sync_copy(hi_hbm.at[wid], idx_i)
    pltpu.sync_copy(pu_hbm.at[wid], par_u)
    pltpu.sync_copy(pi_hbm.at[wid], par_i)
    pltpu.sync_copy(w_hbm, wv)
    pltpu.sync_copy(b_hbm, bv)

    w_regs = [wv[pl.ds(d * LANES, LANES)] for d in range(DBLK)]
    b_vec = bv[...]
    lane = lax.iota(jnp.int32, LANES)
    sems = (sem0, sem1)

    def fire(w):
        slot = w % 2
        return [
            pltpu.async_copy(tab_hbm.at[idx_u.at[w]], rows_u.at[slot], sems[slot]),
            pltpu.async_copy(tab_hbm.at[idx_i.at[w]], rows_i.at[slot], sems[slot]),
        ]

    def user_half(x):
        bits = plsc.bitcast(x, jnp.uint32)
        return plsc.bitcast(bits & HIGH_MASK, jnp.float32)

    def item_half(x):
        bits = plsc.bitcast(x, jnp.uint32)
        return plsc.bitcast(bits << 16, jnp.float32)

    def compute(w):
        slot = w % 2

        @pl.loop(0, GROUPS)
        def _(g):
            pu16 = par_u[w, pl.ds(g * LANES, LANES)]
            pi16 = par_i[w, pl.ds(g * LANES, LANES)]
            res = b_vec
            for k in range(LANES):
                r = g * LANES + k
                off_u = pu16[k] * EMBED_DIM
                off_i = pi16[k] * EMBED_DIM
                acc = (user_half(rows_u[slot, r, pl.ds(off_u, LANES)])
                       * item_half(rows_i[slot, r, pl.ds(off_i, LANES)])
                       * w_regs[0])
                for d in range(1, DBLK):
                    acc = acc + (
                        user_half(rows_u[slot, r, pl.ds(off_u + d * LANES, LANES)])
                        * item_half(rows_i[slot, r, pl.ds(off_i + d * LANES, LANES)])
                        * w_regs[d])
                res = jnp.where(lane == k, res + jnp.sum(acc), res)
            out_v[pl.ds(w * WAVE + g * LANES, LANES)] = res

    pending = fire(0)
    for w in range(NUM_WAVES):
        nxt = fire(w + 1) if w + 1 < NUM_WAVES else []
        for c in pending:
            c.wait()
        compute(w)
        pending = nxt

    pltpu.sync_copy(out_v, out_hbm.at[pl.ds(base, ROWS_PER_WORKER)])


@jax.jit
def kernel(user, item, embed_user_w, embed_item_w, W, b):
    user = user.astype(jnp.int32)
    item = item.astype(jnp.int32)
    shp = (NUM_WORKERS, NUM_WAVES, WAVE)

    def to_packed(r):
        q = r % PACK_BLK
        row = (r // PACK_BLK) * PACK_HALF + (q % PACK_HALF)
        half = q // PACK_HALF
        return row.reshape(shp), half.reshape(shp)

    half_u, par_u = to_packed(user)
    half_i, par_i = to_packed(item)
    tab = _pack_tables(embed_user_w.T, embed_item_w.T)
    w_flat = W.reshape(EMBED_DIM)
    b_pad = jnp.broadcast_to(b, (LANES,))

    mesh = plsc.VectorSubcoreMesh(core_axis_name="c", subcore_axis_name="s")
    cp = pltpu.CompilerParams()
    if "needs_layout_passes" in pltpu.CompilerParams.__dataclass_fields__:
        cp = dataclasses.replace(cp, needs_layout_passes=False)
    run = pl.kernel(
        _gmf_kernel,
        out_type=jax.ShapeDtypeStruct((BATCH,), jnp.float32),
        mesh=mesh,
        compiler_params=cp,
        scratch_types=[
            pltpu.VMEM((NUM_WAVES, WAVE), jnp.int32),
            pltpu.VMEM((NUM_WAVES, WAVE), jnp.int32),
            pltpu.VMEM((NUM_WAVES, WAVE), jnp.int32),
            pltpu.VMEM((NUM_WAVES, WAVE), jnp.int32),
            pltpu.VMEM((2, WAVE, 2 * EMBED_DIM), jnp.float32),
            pltpu.VMEM((2, WAVE, 2 * EMBED_DIM), jnp.float32),
            pltpu.VMEM((EMBED_DIM,), jnp.float32),
            pltpu.VMEM((LANES,), jnp.float32),
            pltpu.VMEM((ROWS_PER_WORKER,), jnp.float32),
            pltpu.SemaphoreType.DMA,
            pltpu.SemaphoreType.DMA,
        ],
    )
    return run(half_u, half_i, par_u, par_i, tab, w_flat, b_pad)


# speedup vs baseline: 3.8770x; 1.0451x over previous
import dataclasses

import jax
import jax.numpy as jnp
import numpy as np
from jax import lax
from jax.experimental import pallas as pl
from jax.experimental.pallas import tpu as pltpu
from jax.experimental.pallas import tpu_sc as plsc

BATCH = 16384
EMBED_DIM = 64
VOCAB = 1000000
NUM_CORES = 2
NUM_SUBCORES = 16
NUM_WORKERS = NUM_CORES * NUM_SUBCORES
ROWS_PER_WORKER = BATCH // NUM_WORKERS
WAVE = 128
NUM_WAVES = ROWS_PER_WORKER // WAVE
LANES = 16
DBLK = EMBED_DIM // LANES
GROUPS = WAVE // LANES
PACK_BLK = 24576
PACK_HALF = PACK_BLK // 2
PACK_STEPS = -(-VOCAB // PACK_BLK)
PACK_ROWS = PACK_STEPS * PACK_HALF
HIGH_MASK = np.uint32(0xFFFF0000)


def _pack_kernel(u_ref, i_ref, o_ref):
    ub = lax.bitcast_convert_type(
        u_ref[...].astype(jnp.bfloat16), jnp.uint16).astype(jnp.uint32)
    ib = lax.bitcast_convert_type(
        i_ref[...].astype(jnp.bfloat16), jnp.uint16).astype(jnp.uint32)
    packed = lax.bitcast_convert_type((ub << 16) | ib, jnp.float32)
    t = packed.T
    o_ref[...] = jnp.concatenate(
        [t[:PACK_HALF], t[PACK_HALF:]], axis=1)


def _pack_tables(ut, it):
    return pl.pallas_call(
        _pack_kernel,
        grid=(PACK_STEPS,),
        in_specs=[
            pl.BlockSpec((EMBED_DIM, PACK_BLK), lambda j: (0, j)),
            pl.BlockSpec((EMBED_DIM, PACK_BLK), lambda j: (0, j)),
        ],
        out_specs=pl.BlockSpec((PACK_HALF, 2 * EMBED_DIM),
                               lambda j: (j, 0)),
        out_shape=jax.ShapeDtypeStruct((PACK_ROWS, 2 * EMBED_DIM),
                                       jnp.float32),
        compiler_params=pltpu.CompilerParams(
            dimension_semantics=("arbitrary",)),
    )(ut, it)


def _gmf_kernel(hu_hbm, hi_hbm, pu_hbm, pi_hbm, tab_hbm, w_hbm, b_hbm,
                out_hbm, idx_u, idx_i, par_u, par_i, rows_u, rows_i, wv, bv,
                out_v, sem0, sem1):
    wid = lax.axis_index("s") * NUM_CORES + lax.axis_index("c")
    base = wid * ROWS_PER_WORKER

    pltpu.sync_copy(hu_hbm.at[wid], idx_u)
    pltpu.sync_copy(hi_hbm.at[wid], idx_i)
    pltpu.sync_copy(pu_hbm.at[wid], par_u)
    pltpu.sync_copy(pi_hbm.at[wid], par_i)
    pltpu.sync_copy(w_hbm, wv)
    pltpu.sync_copy(b_hbm, bv)

    w_regs = [wv[pl.ds(d * LANES, LANES)] for d in range(DBLK)]
    b_vec = bv[...]
    lane = lax.iota(jnp.int32, LANES)
    sems = (sem0, sem1)

    def fire(w):
        slot = w % 2
        return [
            pltpu.async_copy(tab_hbm.at[idx_u.at[w]], rows_u.at[slot], sems[slot]),
            pltpu.async_copy(tab_hbm.at[idx_i.at[w]], rows_i.at[slot], sems[slot]),
        ]

    def user_half(x):
        bits = plsc.bitcast(x, jnp.uint32)
        return plsc.bitcast(bits & HIGH_MASK, jnp.float32)

    def item_half(x):
        bits = plsc.bitcast(x, jnp.uint32)
        return plsc.bitcast(bits << 16, jnp.float32)

    def compute(w):
        slot = w % 2

        @pl.loop(0, GROUPS)
        def _(g):
            pu16 = par_u[w, pl.ds(g * LANES, LANES)]
            pi16 = par_i[w, pl.ds(g * LANES, LANES)]
            res = b_vec
            for k in range(LANES):
                r = g * LANES + k
                off_u = pu16[k] * EMBED_DIM
                off_i = pi16[k] * EMBED_DIM
                acc = (user_half(rows_u[slot, r, pl.ds(off_u, LANES)])
                       * item_half(rows_i[slot, r, pl.ds(off_i, LANES)])
                       * w_regs[0])
                for d in range(1, DBLK):
                    acc = acc + (
                        user_half(rows_u[slot, r, pl.ds(off_u + d * LANES, LANES)])
                        * item_half(rows_i[slot, r, pl.ds(off_i + d * LANES, LANES)])
                        * w_regs[d])
                res = jnp.where(lane == k, res + jnp.sum(acc), res)
            out_v[pl.ds(w * WAVE + g * LANES, LANES)] = res

    pending = fire(0)
    for w in range(NUM_WAVES):
        nxt = fire(w + 1) if w + 1 < NUM_WAVES else []
        for c in pending:
            c.wait()
        compute(w)
        pending = nxt

    pltpu.sync_copy(out_v, out_hbm.at[pl.ds(base, ROWS_PER_WORKER)])


@jax.jit
def kernel(user, item, embed_user_w, embed_item_w, W, b):
    user = user.astype(jnp.int32)
    item = item.astype(jnp.int32)
    shp = (NUM_WORKERS, NUM_WAVES, WAVE)

    def to_packed(r):
        q = r % PACK_BLK
        row = (r // PACK_BLK) * PACK_HALF + (q % PACK_HALF)
        half = q // PACK_HALF
        return row.reshape(shp), half.reshape(shp)

    half_u, par_u = to_packed(user)
    half_i, par_i = to_packed(item)
    tab = _pack_tables(embed_user_w.T, embed_item_w.T)
    w_flat = W.reshape(EMBED_DIM)
    b_pad = jnp.broadcast_to(b, (LANES,))

    mesh = plsc.VectorSubcoreMesh(core_axis_name="c", subcore_axis_name="s")
    cp = pltpu.CompilerParams()
    if "needs_layout_passes" in pltpu.CompilerParams.__dataclass_fields__:
        cp = dataclasses.replace(cp, needs_layout_passes=False)
    run = pl.kernel(
        _gmf_kernel,
        out_type=jax.ShapeDtypeStruct((BATCH,), jnp.float32),
        mesh=mesh,
        compiler_params=cp,
        scratch_types=[
            pltpu.VMEM((NUM_WAVES, WAVE), jnp.int32),
            pltpu.VMEM((NUM_WAVES, WAVE), jnp.int32),
            pltpu.VMEM((NUM_WAVES, WAVE), jnp.int32),
            pltpu.VMEM((NUM_WAVES, WAVE), jnp.int32),
            pltpu.VMEM((2, WAVE, 2 * EMBED_DIM), jnp.float32),
            pltpu.VMEM((2, WAVE, 2 * EMBED_DIM), jnp.float32),
            pltpu.VMEM((EMBED_DIM,), jnp.float32),
            pltpu.VMEM((LANES,), jnp.float32),
            pltpu.VMEM((ROWS_PER_WORKER,), jnp.float32),
            pltpu.SemaphoreType.DMA,
            pltpu.SemaphoreType.DMA,
        ],
    )
    return run(half_u, half_i, par_u, par_i, tab, w_flat, b_pad)
